# Initial kernel scaffold; baseline (speedup 1.0000x reference)
import jax
import jax.numpy as jnp
from jax.experimental import pallas as pl


def kernel(x, edge_index, W1, W2, W_attn):
    raise NotImplementedError("write your pallas kernel here")


# speedup vs baseline: 10.7665x; 10.7665x over previous
import jax
import jax.numpy as jnp
from jax import lax
from jax.experimental import pallas as pl
from jax.experimental.pallas import tpu as pltpu
from jax.experimental.pallas import tpu_sc as plsc

N = 10000
E = 320000
D = 128
HID = 64
C = 32
K = 10
ALPHA = 0.1
LAMBDA_GRAPH = 0.2

NP = 10240
NT = 16
EP = E // NT
CH = 80
NCH = EP // CH
NPT = NP // NT
RBITS = 14
RMASK = (1 << RBITS) - 1


def _mesh():
    return plsc.VectorSubcoreMesh(
        core_axis_name="c", subcore_axis_name="s", num_cores=1,
        num_subcores=NT)


_SC_PARAMS = pltpu.CompilerParams(
    use_tc_tiling_on_sc=False, needs_layout_passes=False)


def _mlp_body(x_ref, w1_ref, w2_ref, o_ref):
    z = jnp.maximum(
        jnp.dot(x_ref[...], w1_ref[...], preferred_element_type=jnp.float32),
        0.0)
    o_ref[...] = jnp.dot(z, w2_ref[...], preferred_element_type=jnp.float32)


def _mlp(x_p, W1, W2):
    nb = 2048
    return pl.pallas_call(
        _mlp_body,
        grid=(NP // nb,),
        in_specs=[
            pl.BlockSpec((nb, D), lambda i: (i, 0)),
            pl.BlockSpec((D, HID), lambda i: (0, 0)),
            pl.BlockSpec((HID, C), lambda i: (0, 0)),
        ],
        out_specs=pl.BlockSpec((nb, C), lambda i: (i, 0)),
        out_shape=jax.ShapeDtypeStruct((NP, C), jnp.float32),
    )(x_p, W1, W2)


def _unpack(pk_v, j, row_c, col_c, need_col=True):
    for g in range(CH // 16):
        v = pk_v[j, pl.ds(g * 16, 16)]
        row_c[pl.ds(g * 16, 16)] = lax.shift_right_logical(v, RBITS)
        if need_col:
            col_c[pl.ds(g * 16, 16)] = lax.bitwise_and(v, RMASK)


def _prop_body(pk_h, L_h, z_h, propa_h, invdeg_h,
               pk_v, row_c, col_c, gbuf, abuf, lbuf, dbuf, ones_v,
               H_s, AH_s, deg_s):
    wid = lax.axis_index("s")
    nb = wid * NPT

    pltpu.sync_copy(pk_h.at[wid], pk_v)
    pltpu.sync_copy(L_h.at[pl.ds(nb, NPT)], lbuf)

    zv = jnp.zeros((16,), jnp.float32)

    def zd(i, carry):
        dbuf[pl.ds(i * 16, 16)] = zv
        return carry
    lax.fori_loop(0, NPT // 16, zd, None)

    for g in range(CH // 16):
        ones_v[pl.ds(g * 16, 16)] = jnp.ones((16,), jnp.float32)

    pltpu.sync_copy(lbuf, H_s.at[pl.ds(nb, NPT)])
    pltpu.sync_copy(z_h, AH_s.at[pl.ds(nb, NPT)])
    pltpu.sync_copy(dbuf, deg_s.at[pl.ds(nb, NPT)])
    pltpu.sync_copy(lbuf, propa_h.at[0, pl.ds(nb, NPT)])
    plsc.subcore_barrier()

    def degj(j, carry):
        _unpack(pk_v, j, row_c, col_c, need_col=False)
        pltpu.sync_copy(ones_v, deg_s.at[row_c], add=True)
        return carry
    lax.fori_loop(0, NCH, degj, None)
    plsc.subcore_barrier()

    pltpu.sync_copy(deg_s.at[pl.ds(nb, NPT)], dbuf)

    def inv16(i, carry):
        v = dbuf[pl.ds(i * 16, 16)]
        dbuf[pl.ds(i * 16, 16)] = 1.0 / jnp.maximum(v, 1.0)
        return carry
    lax.fori_loop(0, NPT // 16, inv16, None)
    pltpu.sync_copy(dbuf, invdeg_h.at[pl.ds(nb, NPT)])

    for k in range(K):
        def prop(j, carry):
            _unpack(pk_v, j, row_c, col_c)
            pltpu.sync_copy(H_s.at[col_c], gbuf)
            pltpu.sync_copy(gbuf, AH_s.at[row_c], add=True)
            return carry
        lax.fori_loop(0, NCH, prop, None)
        plsc.subcore_barrier()

        pltpu.sync_copy(AH_s.at[pl.ds(nb, NPT)], abuf)
        pltpu.sync_copy(z_h, AH_s.at[pl.ds(nb, NPT)])

        def upd(g, carry):
            iv16 = dbuf[pl.ds(g * 16, 16)]
            for e2 in range(16):
                r = g * 16 + e2
                iv = iv16[e2] * (1.0 - ALPHA)
                a0 = abuf[r, pl.ds(0, 16)]
                l0 = lbuf[r, pl.ds(0, 16)]
                abuf[r, pl.ds(0, 16)] = a0 * iv + l0 * ALPHA
                a1 = abuf[r, pl.ds(16, 16)]
                l1 = lbuf[r, pl.ds(16, 16)]
                abuf[r, pl.ds(16, 16)] = a1 * iv + l1 * ALPHA
            return carry
        lax.fori_loop(0, NPT // 16, upd, None)

        pltpu.sync_copy(abuf, H_s.at[pl.ds(nb, NPT)])
        pltpu.sync_copy(abuf, propa_h.at[k + 1, pl.ds(nb, NPT)])
        plsc.subcore_barrier()


def _propagate(pk_t, L_p, zeros_nt):
    kfn = pl.kernel(
        _prop_body,
        out_type=(
            jax.ShapeDtypeStruct((K + 1, NP, C), jnp.float32),
            jax.ShapeDtypeStruct((NP,), jnp.float32),
        ),
        mesh=_mesh(),
        scratch_types=[
            pltpu.VMEM((NCH, CH), jnp.int32),
            pltpu.VMEM((CH,), jnp.int32),
            pltpu.VMEM((CH,), jnp.int32),
            pltpu.VMEM((CH, C), jnp.float32),
            pltpu.VMEM((NPT, C), jnp.float32),
            pltpu.VMEM((NPT, C), jnp.float32),
            pltpu.VMEM((NPT,), jnp.float32),
            pltpu.VMEM((CH,), jnp.float32),
            pltpu.VMEM_SHARED((NP, C), jnp.float32),
            pltpu.VMEM_SHARED((NP, C), jnp.float32),
            pltpu.VMEM_SHARED((NP,), jnp.float32),
        ],
        compiler_params=_SC_PARAMS,
    )
    return kfn(pk_t, L_p, zeros_nt)


def _attn_body(p_ref, w_ref, lo_ref, xn_ref):
    w = w_ref[...]
    attns = []
    for k in range(K + 1):
        attns.append(jnp.sum(p_ref[k] * w, axis=1, keepdims=True))
    attn = jnp.concatenate(attns, axis=1)
    m = jnp.max(attn, axis=1, keepdims=True)
    e = jnp.exp(attn - m)
    s = jnp.sum(e, axis=1, keepdims=True)
    coef = e / s
    out = p_ref[0] * coef[:, 0:1]
    for k in range(1, K + 1):
        out = out + p_ref[k] * coef[:, k:k + 1]
    sq = jnp.sum(out * out, axis=1, keepdims=True)
    norm = jnp.sqrt(jnp.maximum(sq, 1e-10))
    lo_ref[...] = out
    xn_ref[...] = out / norm


def _attention(propa, wa_p):
    nb = 2048
    return pl.pallas_call(
        _attn_body,
        grid=(NP // nb,),
        in_specs=[
            pl.BlockSpec((K + 1, nb, C), lambda i: (0, i, 0)),
            pl.BlockSpec((nb, C), lambda i: (i, 0)),
        ],
        out_specs=[
            pl.BlockSpec((nb, C), lambda i: (i, 0)),
            pl.BlockSpec((nb, C), lambda i: (i, 0)),
        ],
        out_shape=[
            jax.ShapeDtypeStruct((NP, C), jnp.float32),
            jax.ShapeDtypeStruct((NP, C), jnp.float32),
        ],
    )(propa, wa_p)


def _edge_body(pk_h, xn_h, invdeg_h, out_h,
               pk_v, row_c, col_c, bufr, bufc, ex_v, sg, ig, obuf, zb,
               xn_s, s_s, id_s):
    wid = lax.axis_index("s")
    nb = wid * NPT

    pltpu.sync_copy(pk_h.at[wid], pk_v)
    pltpu.sync_copy(xn_h.at[pl.ds(nb, NPT)], xn_s.at[pl.ds(nb, NPT)])
    pltpu.sync_copy(invdeg_h.at[pl.ds(nb, NPT)], id_s.at[pl.ds(nb, NPT)])

    zv = jnp.zeros((16,), jnp.float32)

    def zd(i, carry):
        zb[pl.ds(i * 16, 16)] = zv
        return carry
    lax.fori_loop(0, NPT // 16, zd, None)
    pltpu.sync_copy(zb, s_s.at[pl.ds(nb, NPT)])
    plsc.subcore_barrier()

    def pass1(j, carry):
        _unpack(pk_v, j, row_c, col_c)
        pltpu.sync_copy(xn_s.at[row_c], bufr)
        pltpu.sync_copy(xn_s.at[col_c], bufc)

        def grp(g, c2):
            eidx = g * 16 + lax.iota(jnp.int32, 16)
            acc = jnp.zeros((16,), jnp.float32)
            for c in range(C):
                cv = jnp.full((16,), c, jnp.int32)
                a = plsc.load_gather(bufr, [eidx, cv])
                b = plsc.load_gather(bufc, [eidx, cv])
                acc = acc + a * b
            ex_v[j, pl.ds(g * 16, 16)] = jnp.exp(acc)
            return c2
        lax.fori_loop(0, CH // 16, grp, None)
        pltpu.sync_copy(ex_v.at[j], s_s.at[row_c], add=True)
        return carry
    lax.fori_loop(0, NCH, pass1, None)
    plsc.subcore_barrier()

    def pass2(j, carry):
        _unpack(pk_v, j, row_c, col_c, need_col=False)
        pltpu.sync_copy(s_s.at[row_c], sg)
        pltpu.sync_copy(id_s.at[row_c], ig)

        def grp2(g, c2):
            ex = ex_v[j, pl.ds(g * 16, 16)]
            s = sg[pl.ds(g * 16, 16)]
            iv = ig[pl.ds(g * 16, 16)]
            coef = ex / jnp.maximum(s, 1e-10)
            obuf[j, pl.ds(g * 16, 16)] = (
                iv * (1.0 - LAMBDA_GRAPH) + coef * LAMBDA_GRAPH)
            return c2
        lax.fori_loop(0, CH // 16, grp2, None)
        return carry
    lax.fori_loop(0, NCH, pass2, None)
    pltpu.sync_copy(obuf, out_h.at[wid])


def _edge_softmax(pk_t, xn_p, invdeg):
    kfn = pl.kernel(
        _edge_body,
        out_type=jax.ShapeDtypeStruct((NT, NCH, CH), jnp.float32),
        mesh=_mesh(),
        scratch_types=[
            pltpu.VMEM((NCH, CH), jnp.int32),
            pltpu.VMEM((CH,), jnp.int32),
            pltpu.VMEM((CH,), jnp.int32),
            pltpu.VMEM((CH, C), jnp.float32),
            pltpu.VMEM((CH, C), jnp.float32),
            pltpu.VMEM((NCH, CH), jnp.float32),
            pltpu.VMEM((CH,), jnp.float32),
            pltpu.VMEM((CH,), jnp.float32),
            pltpu.VMEM((NCH, CH), jnp.float32),
            pltpu.VMEM((NPT,), jnp.float32),
            pltpu.VMEM_SHARED((NP, C), jnp.float32),
            pltpu.VMEM_SHARED((NP,), jnp.float32),
            pltpu.VMEM_SHARED((NP,), jnp.float32),
        ],
        compiler_params=_SC_PARAMS,
    )
    return kfn(pk_t, xn_p, invdeg)


def kernel(x, edge_index, W1, W2, W_attn):
    row = edge_index[0].astype(jnp.int32)
    col = edge_index[1].astype(jnp.int32)
    pk_t = ((row << RBITS) | col).reshape(NT, NCH, CH)
    x_p = jnp.pad(x, ((0, NP - N), (0, 0)))
    wa_p = jnp.pad(W_attn, ((0, NP - N), (0, 0)))
    zeros_nt = jnp.zeros((NPT, C), jnp.float32)

    L_p = _mlp(x_p, W1, W2)
    propa, invdeg = _propagate(pk_t, L_p, zeros_nt)
    logits_p, xn_p = _attention(propa, wa_p)
    newadj = _edge_softmax(pk_t, xn_p, invdeg)
    return logits_p[:N], newadj.reshape(E)
